# Initial kernel scaffold; baseline (speedup 1.0000x reference)
import jax
import jax.numpy as jnp
from jax.experimental import pallas as pl


def kernel(input, kernel_cell, kernel_pos, kernel_neg, kernel_elec, W0, b0, W1, b1, W2, b2, W3, b3, Wf, bf):
    raise NotImplementedError("write your pallas kernel here")


# speedup vs baseline: 11.6675x; 11.6675x over previous
import functools

import jax
import jax.numpy as jnp
from jax import lax
from jax.experimental import pallas as pl
from jax.experimental.pallas import tpu as pltpu
from jax.experimental.pallas import tpu_sc as plsc

_NUM_KEYS = 64
_NUM_FEATURES = 128
_WIDTH = 128
_BATCH = 16384

_NC = 2
_NS = 16
_NW = _NC * _NS
_B_PER_W = _BATCH // _NW


def _table_body(kc, kp, kn, ke, w0, b0, w1, b1, w2, b2, w3, b3, wf, bf, out):
    f32 = jnp.float32
    k_col = lax.broadcasted_iota(jnp.int32, (_NUM_KEYS, 1), 0)
    p = k_col % 8
    n = 7 - p
    e = k_col // 8

    f_pos = jnp.zeros((_NUM_KEYS, _NUM_FEATURES), f32)
    f_neg = jnp.zeros((_NUM_KEYS, _NUM_FEATURES), f32)
    f_elec = jnp.zeros((_NUM_KEYS, _NUM_FEATURES), f32)
    for j in range(8):
        f_pos = f_pos + jnp.where(p == j, 1.0, 0.0) * kp[j, :][None, :]
        f_neg = f_neg + jnp.where(n == j, 1.0, 0.0) * kn[j, :][None, :]
        f_elec = f_elec + jnp.where(e == j, 1.0, 0.0) * ke[j, :][None, :]

    h = (
        jnp.dot(f_pos, w0[0:128, :], preferred_element_type=f32)
        + jnp.dot(f_neg, w0[128:256, :], preferred_element_type=f32)
        + jnp.dot(f_elec, w0[256:384, :], preferred_element_type=f32)
        + b0[...]
    )
    h = jnp.maximum(h, 0.0)
    h = jnp.maximum(jnp.dot(h, w1[...], preferred_element_type=f32) + b1[...], 0.0)
    h = jnp.maximum(jnp.dot(h, w2[...], preferred_element_type=f32) + b2[...], 0.0)
    h = jnp.maximum(jnp.dot(h, w3[...], preferred_element_type=f32) + b3[...], 0.0)
    indirect = jnp.dot(h, wf[...], preferred_element_type=f32) + bf[...]

    latent = jnp.where(k_col % 2 == 1, 1.0, 0.0)
    out[...] = latent * kc[...] + (1.0 - latent) * indirect


def _build_table(kc, kp, kn, ke, w0, b0, w1, b1, w2, b2, w3, b3, wf, bf):
    b0r = b0.reshape(1, _WIDTH)
    b1r = b1.reshape(1, _WIDTH)
    b2r = b2.reshape(1, _WIDTH)
    b3r = b3.reshape(1, _WIDTH)
    bfr = bf.reshape(1, _NUM_FEATURES)
    return pl.pallas_call(
        _table_body,
        out_shape=jax.ShapeDtypeStruct((_NUM_KEYS, _NUM_FEATURES), jnp.float32),
    )(kc, kp, kn, ke, w0, b0r, w1, b1r, w2, b2r, w3, b3r, wf, bfr)


def _gather_body(table_hbm, idx_hbm, out_hbm, idx_v, rows_v, sem):
    wid = lax.axis_index("s") * _NC + lax.axis_index("c")
    base = wid * _B_PER_W
    pltpu.sync_copy(idx_hbm.at[pl.ds(base, _B_PER_W)], idx_v)
    pltpu.async_copy(table_hbm.at[idx_v], rows_v, sem).wait()
    pltpu.sync_copy(rows_v, out_hbm.at[pl.ds(base, _B_PER_W)])


def _gather_call(table, idx):
    run = pl.kernel(
        _gather_body,
        mesh=plsc.VectorSubcoreMesh(core_axis_name="c", subcore_axis_name="s",
                                    num_cores=_NC, num_subcores=_NS),
        out_type=jax.ShapeDtypeStruct((_BATCH, _NUM_FEATURES), jnp.float32),
        scratch_types=[
            pltpu.VMEM((_B_PER_W,), jnp.int32),
            pltpu.VMEM((_B_PER_W, _NUM_FEATURES), jnp.float32),
            pltpu.SemaphoreType.DMA,
        ],
    )
    return run(table, idx)


@jax.jit
def kernel(input, kernel_cell, kernel_pos, kernel_neg, kernel_elec,
           W0, b0, W1, b1, W2, b2, W3, b3, Wf, bf):
    table = _build_table(kernel_cell, kernel_pos, kernel_neg, kernel_elec,
                         W0, b0, W1, b1, W2, b2, W3, b3, Wf, bf)
    out = _gather_call(table, input)
    return (out, 0.0)


# speedup vs baseline: 16.3727x; 1.4033x over previous
import functools

import jax
import jax.numpy as jnp
from jax import lax
from jax.experimental import pallas as pl
from jax.experimental.pallas import tpu as pltpu
from jax.experimental.pallas import tpu_sc as plsc

_NUM_KEYS = 64
_NUM_FEATURES = 128
_WIDTH = 128
_BATCH = 16384

_NC = 2
_NS = 16
_NW = _NC * _NS
_B_PER_W = _BATCH // _NW


def _table_body(kc, kp, kn, ke, w0, b0, w1, b1, w2, b2, w3, b3, wf, bf, out):
    f32 = jnp.float32
    k_col = lax.broadcasted_iota(jnp.int32, (_NUM_KEYS, 1), 0)
    p = k_col % 8
    n = 7 - p
    e = k_col // 8

    f_pos = jnp.zeros((_NUM_KEYS, _NUM_FEATURES), f32)
    f_neg = jnp.zeros((_NUM_KEYS, _NUM_FEATURES), f32)
    f_elec = jnp.zeros((_NUM_KEYS, _NUM_FEATURES), f32)
    for j in range(8):
        f_pos = f_pos + jnp.where(p == j, 1.0, 0.0) * kp[j, :][None, :]
        f_neg = f_neg + jnp.where(n == j, 1.0, 0.0) * kn[j, :][None, :]
        f_elec = f_elec + jnp.where(e == j, 1.0, 0.0) * ke[j, :][None, :]

    h = (
        jnp.dot(f_pos, w0[0:128, :], preferred_element_type=f32)
        + jnp.dot(f_neg, w0[128:256, :], preferred_element_type=f32)
        + jnp.dot(f_elec, w0[256:384, :], preferred_element_type=f32)
        + b0[...]
    )
    h = jnp.maximum(h, 0.0)
    h = jnp.maximum(jnp.dot(h, w1[...], preferred_element_type=f32) + b1[...], 0.0)
    h = jnp.maximum(jnp.dot(h, w2[...], preferred_element_type=f32) + b2[...], 0.0)
    h = jnp.maximum(jnp.dot(h, w3[...], preferred_element_type=f32) + b3[...], 0.0)
    indirect = jnp.dot(h, wf[...], preferred_element_type=f32) + bf[...]

    latent = jnp.where(k_col % 2 == 1, 1.0, 0.0)
    out[...] = latent * kc[...] + (1.0 - latent) * indirect


def _build_table(kc, kp, kn, ke, w0, b0, w1, b1, w2, b2, w3, b3, wf, bf):
    b0r = b0.reshape(1, _WIDTH)
    b1r = b1.reshape(1, _WIDTH)
    b2r = b2.reshape(1, _WIDTH)
    b3r = b3.reshape(1, _WIDTH)
    bfr = bf.reshape(1, _NUM_FEATURES)
    return pl.pallas_call(
        _table_body,
        out_shape=jax.ShapeDtypeStruct((_NUM_KEYS, _NUM_FEATURES), jnp.float32),
    )(kc, kp, kn, ke, w0, b0r, w1, b1r, w2, b2r, w3, b3r, wf, bfr)


def _gather_body(table_hbm, idx_hbm, out_hbm, table_sh, idx_v, rows_v, sem):
    sid = lax.axis_index("s")
    wid = sid * _NC + lax.axis_index("c")
    base = wid * _B_PER_W

    @pl.when(sid == 0)
    def _():
        pltpu.sync_copy(table_hbm, table_sh)

    pltpu.sync_copy(idx_hbm.at[pl.ds(base, _B_PER_W)], idx_v)
    plsc.subcore_barrier()
    pltpu.async_copy(table_sh.at[idx_v], rows_v, sem).wait()
    pltpu.sync_copy(rows_v, out_hbm.at[pl.ds(base, _B_PER_W)])


def _gather_call(table, idx):
    run = pl.kernel(
        _gather_body,
        mesh=plsc.VectorSubcoreMesh(core_axis_name="c", subcore_axis_name="s",
                                    num_cores=_NC, num_subcores=_NS),
        out_type=jax.ShapeDtypeStruct((_BATCH, _NUM_FEATURES), jnp.float32),
        scratch_types=[
            pltpu.VMEM_SHARED((_NUM_KEYS, _NUM_FEATURES), jnp.float32),
            pltpu.VMEM((_B_PER_W,), jnp.int32),
            pltpu.VMEM((_B_PER_W, _NUM_FEATURES), jnp.float32),
            pltpu.SemaphoreType.DMA,
        ],
    )
    return run(table, idx)


@jax.jit
def kernel(input, kernel_cell, kernel_pos, kernel_neg, kernel_elec,
           W0, b0, W1, b1, W2, b2, W3, b3, Wf, bf):
    table = _build_table(kernel_cell, kernel_pos, kernel_neg, kernel_elec,
                         W0, b0, W1, b1, W2, b2, W3, b3, Wf, bf)
    out = _gather_call(table, input)
    return (out, 0.0)


# speedup vs baseline: 16.9879x; 1.0376x over previous
import functools

import jax
import jax.numpy as jnp
from jax import lax
from jax.experimental import pallas as pl
from jax.experimental.pallas import tpu as pltpu
from jax.experimental.pallas import tpu_sc as plsc

_NUM_KEYS = 64
_NUM_FEATURES = 128
_WIDTH = 128
_BATCH = 16384

_NC = 2
_NS = 16
_NW = _NC * _NS
_B_PER_W = _BATCH // _NW


def _table_body(kc, kp, kn, ke, w0, b0, w1, b1, w2, b2, w3, b3, wf, bf, out):
    f32 = jnp.float32
    k_col = lax.broadcasted_iota(jnp.int32, (_NUM_KEYS, 1), 0)
    p = k_col % 8
    n = 7 - p
    e = k_col // 8

    f_pos = jnp.zeros((_NUM_KEYS, _NUM_FEATURES), f32)
    f_neg = jnp.zeros((_NUM_KEYS, _NUM_FEATURES), f32)
    f_elec = jnp.zeros((_NUM_KEYS, _NUM_FEATURES), f32)
    for j in range(8):
        f_pos = f_pos + jnp.where(p == j, 1.0, 0.0) * kp[j, :][None, :]
        f_neg = f_neg + jnp.where(n == j, 1.0, 0.0) * kn[j, :][None, :]
        f_elec = f_elec + jnp.where(e == j, 1.0, 0.0) * ke[j, :][None, :]

    h = (
        jnp.dot(f_pos, w0[0:128, :], preferred_element_type=f32)
        + jnp.dot(f_neg, w0[128:256, :], preferred_element_type=f32)
        + jnp.dot(f_elec, w0[256:384, :], preferred_element_type=f32)
        + b0[...]
    )
    h = jnp.maximum(h, 0.0)
    h = jnp.maximum(jnp.dot(h, w1[...], preferred_element_type=f32) + b1[...], 0.0)
    h = jnp.maximum(jnp.dot(h, w2[...], preferred_element_type=f32) + b2[...], 0.0)
    h = jnp.maximum(jnp.dot(h, w3[...], preferred_element_type=f32) + b3[...], 0.0)
    indirect = jnp.dot(h, wf[...], preferred_element_type=f32) + bf[...]

    latent = jnp.where(k_col % 2 == 1, 1.0, 0.0)
    out[...] = latent * kc[...] + (1.0 - latent) * indirect


def _build_table(kc, kp, kn, ke, w0, b0, w1, b1, w2, b2, w3, b3, wf, bf):
    b0r = b0.reshape(1, _WIDTH)
    b1r = b1.reshape(1, _WIDTH)
    b2r = b2.reshape(1, _WIDTH)
    b3r = b3.reshape(1, _WIDTH)
    bfr = bf.reshape(1, _NUM_FEATURES)
    return pl.pallas_call(
        _table_body,
        out_shape=jax.ShapeDtypeStruct((_NUM_KEYS, _NUM_FEATURES), jnp.float32),
    )(kc, kp, kn, ke, w0, b0r, w1, b1r, w2, b2r, w3, b3r, wf, bfr)


_N_CHUNKS = 4
_CHUNK = _B_PER_W // _N_CHUNKS


def _gather_body(table_hbm, idx_hbm, out_hbm, table_sh, idx_v, rows_v,
                 gsem, ssem):
    sid = lax.axis_index("s")
    wid = sid * _NC + lax.axis_index("c")
    base = wid * _B_PER_W

    @pl.when(sid == 0)
    def _():
        pltpu.sync_copy(table_hbm, table_sh)

    pltpu.sync_copy(idx_hbm.at[pl.ds(base, _B_PER_W)], idx_v)
    plsc.subcore_barrier()

    gathers = []
    for i in range(_N_CHUNKS):
        gathers.append(pltpu.async_copy(
            table_sh.at[idx_v.at[pl.ds(i * _CHUNK, _CHUNK)]],
            rows_v.at[pl.ds(i * _CHUNK, _CHUNK)],
            gsem,
        ))
    scatters = []
    for i in range(_N_CHUNKS):
        gathers[i].wait()
        scatters.append(pltpu.async_copy(
            rows_v.at[pl.ds(i * _CHUNK, _CHUNK)],
            out_hbm.at[pl.ds(base + i * _CHUNK, _CHUNK)],
            ssem,
        ))
    for s in scatters:
        s.wait()


def _gather_call(table, idx):
    run = pl.kernel(
        _gather_body,
        mesh=plsc.VectorSubcoreMesh(core_axis_name="c", subcore_axis_name="s",
                                    num_cores=_NC, num_subcores=_NS),
        out_type=jax.ShapeDtypeStruct((_BATCH, _NUM_FEATURES), jnp.float32),
        scratch_types=[
            pltpu.VMEM_SHARED((_NUM_KEYS, _NUM_FEATURES), jnp.float32),
            pltpu.VMEM((_B_PER_W,), jnp.int32),
            pltpu.VMEM((_B_PER_W, _NUM_FEATURES), jnp.float32),
            pltpu.SemaphoreType.DMA,
            pltpu.SemaphoreType.DMA,
        ],
    )
    return run(table, idx)


@jax.jit
def kernel(input, kernel_cell, kernel_pos, kernel_neg, kernel_elec,
           W0, b0, W1, b1, W2, b2, W3, b3, Wf, bf):
    table = _build_table(kernel_cell, kernel_pos, kernel_neg, kernel_elec,
                         W0, b0, W1, b1, W2, b2, W3, b3, Wf, bf)
    out = _gather_call(table, input)
    return (out, 0.0)


# speedup vs baseline: 17.0066x; 1.0011x over previous
import functools

import jax
import jax.numpy as jnp
from jax import lax
from jax.experimental import pallas as pl
from jax.experimental.pallas import tpu as pltpu
from jax.experimental.pallas import tpu_sc as plsc

_NUM_KEYS = 64
_NUM_FEATURES = 128
_WIDTH = 128
_BATCH = 16384

_NC = 2
_NS = 16
_NW = _NC * _NS
_B_PER_W = _BATCH // _NW


def _table_body(kc, kp, kn, ke, w0, b0, w1, b1, w2, b2, w3, b3, wf, bf, out):
    f32 = jnp.float32
    k_col = lax.broadcasted_iota(jnp.int32, (_NUM_KEYS, 1), 0)
    p = k_col % 8
    n = 7 - p
    e = k_col // 8

    f_pos = jnp.zeros((_NUM_KEYS, _NUM_FEATURES), f32)
    f_neg = jnp.zeros((_NUM_KEYS, _NUM_FEATURES), f32)
    f_elec = jnp.zeros((_NUM_KEYS, _NUM_FEATURES), f32)
    for j in range(8):
        f_pos = f_pos + jnp.where(p == j, 1.0, 0.0) * kp[j, :][None, :]
        f_neg = f_neg + jnp.where(n == j, 1.0, 0.0) * kn[j, :][None, :]
        f_elec = f_elec + jnp.where(e == j, 1.0, 0.0) * ke[j, :][None, :]

    h = (
        jnp.dot(f_pos, w0[0:128, :], preferred_element_type=f32)
        + jnp.dot(f_neg, w0[128:256, :], preferred_element_type=f32)
        + jnp.dot(f_elec, w0[256:384, :], preferred_element_type=f32)
        + b0[...]
    )
    h = jnp.maximum(h, 0.0)
    h = jnp.maximum(jnp.dot(h, w1[...], preferred_element_type=f32) + b1[...], 0.0)
    h = jnp.maximum(jnp.dot(h, w2[...], preferred_element_type=f32) + b2[...], 0.0)
    h = jnp.maximum(jnp.dot(h, w3[...], preferred_element_type=f32) + b3[...], 0.0)
    indirect = jnp.dot(h, wf[...], preferred_element_type=f32) + bf[...]

    latent = jnp.where(k_col % 2 == 1, 1.0, 0.0)
    out[...] = latent * kc[...] + (1.0 - latent) * indirect


def _build_table(kc, kp, kn, ke, w0, b0, w1, b1, w2, b2, w3, b3, wf, bf):
    b0r = b0.reshape(1, _WIDTH)
    b1r = b1.reshape(1, _WIDTH)
    b2r = b2.reshape(1, _WIDTH)
    b3r = b3.reshape(1, _WIDTH)
    bfr = bf.reshape(1, _NUM_FEATURES)
    return pl.pallas_call(
        _table_body,
        out_shape=jax.ShapeDtypeStruct((_NUM_KEYS, _NUM_FEATURES), jnp.float32),
    )(kc, kp, kn, ke, w0, b0r, w1, b1r, w2, b2r, w3, b3r, wf, bfr)


_N_CHUNKS = 8
_CHUNK = _B_PER_W // _N_CHUNKS


def _gather_body(table_hbm, idx_hbm, out_hbm, table_sh, idx_v, rows_v,
                 gsem, ssem):
    sid = lax.axis_index("s")
    wid = sid * _NC + lax.axis_index("c")
    base = wid * _B_PER_W

    @pl.when(sid == 0)
    def _():
        pltpu.sync_copy(table_hbm, table_sh)

    pltpu.sync_copy(idx_hbm.at[pl.ds(base, _B_PER_W)], idx_v)
    plsc.subcore_barrier()

    gathers = []
    for i in range(_N_CHUNKS):
        gathers.append(pltpu.async_copy(
            table_sh.at[idx_v.at[pl.ds(i * _CHUNK, _CHUNK)]],
            rows_v.at[pl.ds(i * _CHUNK, _CHUNK)],
            gsem,
        ))
    scatters = []
    for i in range(_N_CHUNKS):
        gathers[i].wait()
        scatters.append(pltpu.async_copy(
            rows_v.at[pl.ds(i * _CHUNK, _CHUNK)],
            out_hbm.at[pl.ds(base + i * _CHUNK, _CHUNK)],
            ssem,
        ))
    for s in scatters:
        s.wait()


def _gather_call(table, idx):
    run = pl.kernel(
        _gather_body,
        mesh=plsc.VectorSubcoreMesh(core_axis_name="c", subcore_axis_name="s",
                                    num_cores=_NC, num_subcores=_NS),
        out_type=jax.ShapeDtypeStruct((_BATCH, _NUM_FEATURES), jnp.float32),
        scratch_types=[
            pltpu.VMEM_SHARED((_NUM_KEYS, _NUM_FEATURES), jnp.float32),
            pltpu.VMEM((_B_PER_W,), jnp.int32),
            pltpu.VMEM((_B_PER_W, _NUM_FEATURES), jnp.float32),
            pltpu.SemaphoreType.DMA,
            pltpu.SemaphoreType.DMA,
        ],
    )
    return run(table, idx)


@jax.jit
def kernel(input, kernel_cell, kernel_pos, kernel_neg, kernel_elec,
           W0, b0, W1, b1, W2, b2, W3, b3, Wf, bf):
    table = _build_table(kernel_cell, kernel_pos, kernel_neg, kernel_elec,
                         W0, b0, W1, b1, W2, b2, W3, b3, Wf, bf)
    out = _gather_call(table, input)
    return (out, 0.0)


# speedup vs baseline: 17.0314x; 1.0015x over previous
import jax
import jax.numpy as jnp
from jax import lax
from jax.experimental import pallas as pl
from jax.experimental.pallas import tpu as pltpu
from jax.experimental.pallas import tpu_sc as plsc

_NUM_KEYS = 64
_NUM_FEATURES = 128
_WIDTH = 128
_BATCH = 16384

_NC = 2
_NS = 16
_NW = _NC * _NS
_B_PER_W = _BATCH // _NW


def _table_body(kc, kp, kn, ke, w0, b0, w1, b1, w2, b2, w3, b3, wf, bf, out):
    f32 = jnp.float32
    k_col = lax.broadcasted_iota(jnp.int32, (_NUM_KEYS, 1), 0)
    p = k_col % 8
    n = 7 - p
    e = k_col // 8

    f_pos = jnp.zeros((_NUM_KEYS, _NUM_FEATURES), f32)
    f_neg = jnp.zeros((_NUM_KEYS, _NUM_FEATURES), f32)
    f_elec = jnp.zeros((_NUM_KEYS, _NUM_FEATURES), f32)
    for j in range(8):
        f_pos = f_pos + jnp.where(p == j, 1.0, 0.0) * kp[j, :][None, :]
        f_neg = f_neg + jnp.where(n == j, 1.0, 0.0) * kn[j, :][None, :]
        f_elec = f_elec + jnp.where(e == j, 1.0, 0.0) * ke[j, :][None, :]

    h = (
        jnp.dot(f_pos, w0[0:128, :], preferred_element_type=f32)
        + jnp.dot(f_neg, w0[128:256, :], preferred_element_type=f32)
        + jnp.dot(f_elec, w0[256:384, :], preferred_element_type=f32)
        + b0[...]
    )
    h = jnp.maximum(h, 0.0)
    h = jnp.maximum(jnp.dot(h, w1[...], preferred_element_type=f32) + b1[...], 0.0)
    h = jnp.maximum(jnp.dot(h, w2[...], preferred_element_type=f32) + b2[...], 0.0)
    h = jnp.maximum(jnp.dot(h, w3[...], preferred_element_type=f32) + b3[...], 0.0)
    indirect = jnp.dot(h, wf[...], preferred_element_type=f32) + bf[...]

    latent = jnp.where(k_col % 2 == 1, 1.0, 0.0)
    out[...] = latent * kc[...] + (1.0 - latent) * indirect


def _build_table(kc, kp, kn, ke, w0, b0, w1, b1, w2, b2, w3, b3, wf, bf):
    b0r = b0.reshape(1, _WIDTH)
    b1r = b1.reshape(1, _WIDTH)
    b2r = b2.reshape(1, _WIDTH)
    b3r = b3.reshape(1, _WIDTH)
    bfr = bf.reshape(1, _NUM_FEATURES)
    return pl.pallas_call(
        _table_body,
        out_shape=jax.ShapeDtypeStruct((_NUM_KEYS, _NUM_FEATURES), jnp.float32),
    )(kc, kp, kn, ke, w0, b0r, w1, b1r, w2, b2r, w3, b3r, wf, bfr)


_N_CHUNKS = 8
_CHUNK = _B_PER_W // _N_CHUNKS


def _gather_body(table_hbm, idx_hbm, out_hbm, table_sh, idx_v, rows_v,
                 gsem, ssem):
    sid = lax.axis_index("s")
    wid = sid * _NC + lax.axis_index("c")
    base = wid * _B_PER_W

    @pl.when(sid == 0)
    def _():
        pltpu.sync_copy(table_hbm, table_sh)

    pltpu.sync_copy(idx_hbm.at[pl.ds(base, _B_PER_W)], idx_v)
    plsc.subcore_barrier()

    gathers = []
    for i in range(_N_CHUNKS):
        gathers.append(pltpu.async_copy(
            table_sh.at[idx_v.at[pl.ds(i * _CHUNK, _CHUNK)]],
            rows_v.at[pl.ds(i * _CHUNK, _CHUNK)],
            gsem,
        ))
    scatters = []
    for i in range(_N_CHUNKS):
        gathers[i].wait()
        scatters.append(pltpu.async_copy(
            rows_v.at[pl.ds(i * _CHUNK, _CHUNK)],
            out_hbm.at[pl.ds(base + i * _CHUNK, _CHUNK)],
            ssem,
        ))
    for s in scatters:
        s.wait()


def _gather_call(table, idx):
    run = pl.kernel(
        _gather_body,
        mesh=plsc.VectorSubcoreMesh(core_axis_name="c", subcore_axis_name="s",
                                    num_cores=_NC, num_subcores=_NS),
        out_type=jax.ShapeDtypeStruct((_BATCH, _NUM_FEATURES), jnp.float32),
        scratch_types=[
            pltpu.VMEM_SHARED((_NUM_KEYS, _NUM_FEATURES), jnp.float32),
            pltpu.VMEM((_B_PER_W,), jnp.int32),
            pltpu.VMEM((_B_PER_W, _NUM_FEATURES), jnp.float32),
            pltpu.SemaphoreType.DMA,
            pltpu.SemaphoreType.DMA,
        ],
    )
    return run(table, idx)


@jax.jit
def kernel(input, kernel_cell, kernel_pos, kernel_neg, kernel_elec,
           W0, b0, W1, b1, W2, b2, W3, b3, Wf, bf):
    table = _build_table(kernel_cell, kernel_pos, kernel_neg, kernel_elec,
                         W0, b0, W1, b1, W2, b2, W3, b3, Wf, bf)
    out = _gather_call(table, input)
    return (out, 0.0)
